# baseline (speedup 1.0000x reference)
import jax
import jax.numpy as jnp
from jax.experimental import pallas as pl

_B = 16
_N = 4096
_GRID = 1
_BB = _B // _GRID
_AR = _B * _N // 128
_ARB = _AR // _GRID


def _body(x_ref, out_ref, cnt_ref, assoc_ref):
    i = pl.program_id(0)
    y = x_ref[...] * 0.5
    coord = jax.lax.broadcasted_iota(jnp.int32, (_BB, 4, _N), 1)
    out_ref[...] = jnp.where(coord < 2, jnp.floor(y), jnp.ceil(y))
    r = jax.lax.broadcasted_iota(jnp.int32, (_ARB, 128), 0)
    assoc_ref[...] = (r + i * _ARB) >> 5
    cnt_ref[...] = jnp.full((16,), _N, dtype=jnp.int32)


_tc_call = pl.pallas_call(
    _body,
    grid=(_GRID,),
    in_specs=[pl.BlockSpec((_BB, 4, _N), lambda i: (i, 0, 0))],
    out_specs=[
        pl.BlockSpec((_BB, 4, _N), lambda i: (i, 0, 0)),
        pl.BlockSpec((16,), lambda i: (0,)),
        pl.BlockSpec((_ARB, 128), lambda i: (i, 0)),
    ],
    out_shape=[
        jax.ShapeDtypeStruct((_B, 4, _N), jnp.float32),
        jax.ShapeDtypeStruct((16,), jnp.int32),
        jax.ShapeDtypeStruct((_AR, 128), jnp.int32),
    ],
)


def kernel(bbox_batch):
    xt = bbox_batch.transpose(0, 2, 1)
    out_t, cnt, assoc = _tc_call(xt)
    return (
        out_t.transpose(0, 2, 1).reshape(_B * _N, 4),
        cnt,
        assoc.reshape(_B * _N),
    )


# speedup vs baseline: 1.0366x; 1.0366x over previous
import jax
import jax.numpy as jnp
from jax.experimental import pallas as pl
from jax.experimental.pallas import tpu as pltpu

_B = 16
_N = 4096
_GRID = 2
_BB = _B // _GRID
_AR = _B * _N // 128
_ARB = _AR // _GRID


def _body(x_ref, out_ref, cnt_ref, assoc_ref):
    i = pl.program_id(0)
    y = x_ref[pl.ds(i * _BB, _BB)] * 0.5
    coord = jax.lax.broadcasted_iota(jnp.int32, (_BB, 4, _N), 1)
    out_ref[...] = jnp.where(coord < 2, jnp.floor(y), jnp.ceil(y))
    r = jax.lax.broadcasted_iota(jnp.int32, (_ARB, 128), 0)
    assoc_ref[...] = (r + i * _ARB) >> 5
    cnt_ref[...] = jnp.full((16,), _N, dtype=jnp.int32)


_tc_call = pl.pallas_call(
    _body,
    grid=(_GRID,),
    in_specs=[pl.BlockSpec(memory_space=pltpu.VMEM)],
    out_specs=[
        pl.BlockSpec((_BB, 4, _N), lambda i: (i, 0, 0)),
        pl.BlockSpec((16,), lambda i: (0,)),
        pl.BlockSpec((_ARB, 128), lambda i: (i, 0)),
    ],
    out_shape=[
        jax.ShapeDtypeStruct((_B, 4, _N), jnp.float32),
        jax.ShapeDtypeStruct((16,), jnp.int32),
        jax.ShapeDtypeStruct((_AR, 128), jnp.int32),
    ],
)


def kernel(bbox_batch):
    xt = bbox_batch.transpose(0, 2, 1)
    out_t, cnt, assoc = _tc_call(xt)
    return (
        out_t.transpose(0, 2, 1).reshape(_B * _N, 4),
        cnt,
        assoc.reshape(_B * _N),
    )


# speedup vs baseline: 1.1162x; 1.0768x over previous
import jax
import jax.numpy as jnp
from jax.experimental import pallas as pl

_B = 16
_N = 4096
_GRID = 2
_BB = _B // _GRID
_AR = _B * _N // 128
_ARB = _AR // _GRID


def _body(x_ref, out_ref, cnt_ref, assoc_ref):
    i = pl.program_id(0)
    y = x_ref[...] * 0.5
    coord = jax.lax.broadcasted_iota(jnp.int32, (_BB, 4, _N), 1)
    out_ref[...] = jnp.where(coord < 2, jnp.floor(y), jnp.ceil(y))
    r = jax.lax.broadcasted_iota(jnp.int32, (_ARB, 128), 0)
    assoc_ref[...] = (r + i * _ARB) >> 5
    cnt_ref[...] = jnp.full((16,), _N, dtype=jnp.int32)


_tc_call = pl.pallas_call(
    _body,
    grid=(_GRID,),
    in_specs=[pl.BlockSpec((_BB, 4, _N), lambda i: (i, 0, 0))],
    out_specs=[
        pl.BlockSpec((_BB, 4, _N), lambda i: (i, 0, 0)),
        pl.BlockSpec((16,), lambda i: (0,)),
        pl.BlockSpec((_ARB, 128), lambda i: (i, 0)),
    ],
    out_shape=[
        jax.ShapeDtypeStruct((_B, 4, _N), jnp.float32),
        jax.ShapeDtypeStruct((16,), jnp.int32),
        jax.ShapeDtypeStruct((_AR, 128), jnp.int32),
    ],
)


def kernel(bbox_batch):
    xt = bbox_batch.transpose(0, 2, 1)
    out_t, cnt, assoc = _tc_call(xt)
    return (
        out_t.transpose(0, 2, 1).reshape(_B * _N, 4),
        cnt,
        assoc.reshape(_B * _N),
    )
